# initial kernel scaffold (unmeasured)
import jax
import jax.numpy as jnp
from jax import lax
from jax.experimental import pallas as pl
from jax.experimental.pallas import tpu as pltpu

N_DEV = 4


def kernel(x, w_mat, scale_x, scale_w):
    m_per, k = x.shape
    n_per = w_mat.shape[1]

    def body(x_ref, w_ref, sx_ref, sw_ref, out_ref, xg_ref, send_sems, recv_sems):
        my = lax.axis_index("i")
        left = (my - 1) % N_DEV
        right = (my + 1) % N_DEV

        barrier_sem = pltpu.get_barrier_semaphore()
        for nbr in [left, right]:
            pl.semaphore_signal(
                barrier_sem, inc=1,
                device_id=(nbr,), device_id_type=pl.DeviceIdType.MESH,
            )
        pl.semaphore_wait(barrier_sem, 2)

        scale = sx_ref[0] * sw_ref[0]

        def compute(origin):
            acc = jax.lax.dot_general(
                xg_ref[origin], w_ref[...],
                (((1,), (0,)), ((), ())),
                preferred_element_type=jnp.int32,
            )
            y = acc.astype(jnp.float32) * scale
            out_ref[pl.ds(origin * m_per, m_per), :] = y * (
                1.0 / (1.0 + jnp.exp(-y))
            )

        xg_ref[my] = x_ref[...]

        for h in range(N_DEV - 1):
            send_origin = (my - h) % N_DEV
            recv_origin = (my - h - 1) % N_DEV
            rdma = pltpu.make_async_remote_copy(
                src_ref=xg_ref.at[send_origin],
                dst_ref=xg_ref.at[send_origin],
                send_sem=send_sems.at[h],
                recv_sem=recv_sems.at[h],
                device_id=(right,),
                device_id_type=pl.DeviceIdType.MESH,
            )
            rdma.start()
            compute(send_origin)
            rdma.wait()

        compute((my + 1) % N_DEV)

    return pl.pallas_call(
        body,
        out_shape=jax.ShapeDtypeStruct((N_DEV * m_per, n_per), jnp.float32),
        in_specs=[
            pl.BlockSpec(memory_space=pltpu.VMEM),
            pl.BlockSpec(memory_space=pltpu.VMEM),
            pl.BlockSpec(memory_space=pltpu.SMEM),
            pl.BlockSpec(memory_space=pltpu.SMEM),
        ],
        out_specs=pl.BlockSpec(memory_space=pltpu.VMEM),
        scratch_shapes=[
            pltpu.VMEM((N_DEV, m_per, k), jnp.int8),
            pltpu.SemaphoreType.DMA((N_DEV - 1,)),
            pltpu.SemaphoreType.DMA((N_DEV - 1,)),
        ],
        compiler_params=pltpu.CompilerParams(collective_id=0),
    )(x, w_mat, scale_x, scale_w)


# baseline (device time: 194195 ns/iter reference)
import jax
import jax.numpy as jnp
from jax import lax
from jax.experimental import pallas as pl
from jax.experimental.pallas import tpu as pltpu

N_DEV = 4


def kernel(x, w_mat, scale_x, scale_w):
    m_per, k = x.shape
    n_per = w_mat.shape[1]

    def body(x_ref, w_ref, sx_ref, sw_ref, out_ref, xg_ref, stage_ref,
             send_sems, recv_sems, copy_sem):
        my = lax.axis_index("i")
        left = (my - 1) % N_DEV
        right = (my + 1) % N_DEV

        barrier_sem = pltpu.get_barrier_semaphore()
        for nbr in [left, right]:
            pl.semaphore_signal(
                barrier_sem, inc=1,
                device_id=(nbr,), device_id_type=pl.DeviceIdType.MESH,
            )
        pl.semaphore_wait(barrier_sem, 2)

        scale = sx_ref[0] * sw_ref[0]

        def compute(origin):
            acc = jax.lax.dot_general(
                xg_ref[origin], w_ref[...],
                (((1,), (0,)), ((), ())),
                preferred_element_type=jnp.int32,
            )
            y = acc.astype(jnp.float32) * scale
            stage_ref[...] = y * (1.0 / (1.0 + jnp.exp(-y)))
            copy = pltpu.make_async_copy(
                stage_ref,
                out_ref.at[pl.ds(origin * m_per, m_per), :],
                copy_sem,
            )
            copy.start()
            copy.wait()

        xg_ref[my] = x_ref[...]

        for h in range(N_DEV - 1):
            send_origin = (my - h) % N_DEV
            recv_origin = (my - h - 1) % N_DEV
            rdma = pltpu.make_async_remote_copy(
                src_ref=xg_ref.at[send_origin],
                dst_ref=xg_ref.at[send_origin],
                send_sem=send_sems.at[h],
                recv_sem=recv_sems.at[h],
                device_id=(right,),
                device_id_type=pl.DeviceIdType.MESH,
            )
            rdma.start()
            compute(send_origin)
            rdma.wait()

        compute((my + 1) % N_DEV)

    return pl.pallas_call(
        body,
        out_shape=jax.ShapeDtypeStruct((N_DEV * m_per, n_per), jnp.float32),
        in_specs=[
            pl.BlockSpec(memory_space=pltpu.VMEM),
            pl.BlockSpec(memory_space=pltpu.VMEM),
            pl.BlockSpec(memory_space=pltpu.SMEM),
            pl.BlockSpec(memory_space=pltpu.SMEM),
        ],
        out_specs=pl.BlockSpec(memory_space=pl.ANY),
        scratch_shapes=[
            pltpu.VMEM((N_DEV, m_per, k), jnp.int8),
            pltpu.VMEM((m_per, n_per), jnp.float32),
            pltpu.SemaphoreType.DMA((N_DEV - 1,)),
            pltpu.SemaphoreType.DMA((N_DEV - 1,)),
            pltpu.SemaphoreType.DMA,
        ],
        compiler_params=pltpu.CompilerParams(
            collective_id=0,
            vmem_limit_bytes=100 * 1024 * 1024,
        ),
    )(x, w_mat, scale_x, scale_w)


# device time: 125430 ns/iter; 1.5482x vs baseline; 1.5482x over previous
import jax
import jax.numpy as jnp
from jax import lax
from jax.experimental import pallas as pl
from jax.experimental.pallas import tpu as pltpu

N_DEV = 4


def kernel(x, w_mat, scale_x, scale_w):
    m_per, k = x.shape
    n_per = w_mat.shape[1]
    m_half = m_per // 2

    def body(x_ref, w_ref, sx_ref, sw_ref, out_ref, xg_ref, stage_ref,
             send_r, recv_r, send_l, recv_l, copy_sems):
        my = lax.axis_index("i")
        left = (my - 1) % N_DEV
        right = (my + 1) % N_DEV

        barrier_sem = pltpu.get_barrier_semaphore()
        for nbr in [left, right]:
            pl.semaphore_signal(
                barrier_sem, inc=1,
                device_id=(nbr,), device_id_type=pl.DeviceIdType.MESH,
            )
        pl.semaphore_wait(barrier_sem, 2)

        scale = sx_ref[0] * sw_ref[0]

        pending = [None, None]
        next_slot = [0]

        def compute_half(origin, top):
            row0 = 0 if top else m_half
            s = next_slot[0]
            next_slot[0] = 1 - s
            if pending[s] is not None:
                pending[s].wait()
            acc = jax.lax.dot_general(
                xg_ref[origin, pl.ds(row0, m_half), :], w_ref[...],
                (((1,), (0,)), ((), ())),
                preferred_element_type=jnp.int32,
            )
            y = acc.astype(jnp.float32) * scale
            stage_ref[s] = y * (1.0 / (1.0 + jnp.exp(-y)))
            copy = pltpu.make_async_copy(
                stage_ref.at[s],
                out_ref.at[pl.ds(origin * m_per + row0, m_half), :],
                copy_sems.at[s],
            )
            copy.start()
            pending[s] = copy

        xg_ref[my] = x_ref[...]

        for h in range(N_DEV - 1):
            r_origin = (my - h) % N_DEV
            l_origin = (my + h) % N_DEV
            r = pltpu.make_async_remote_copy(
                src_ref=xg_ref.at[r_origin, pl.ds(0, m_half), :],
                dst_ref=xg_ref.at[r_origin, pl.ds(0, m_half), :],
                send_sem=send_r.at[h],
                recv_sem=recv_r.at[h],
                device_id=(right,),
                device_id_type=pl.DeviceIdType.MESH,
            )
            l = pltpu.make_async_remote_copy(
                src_ref=xg_ref.at[l_origin, pl.ds(m_half, m_half), :],
                dst_ref=xg_ref.at[l_origin, pl.ds(m_half, m_half), :],
                send_sem=send_l.at[h],
                recv_sem=recv_l.at[h],
                device_id=(left,),
                device_id_type=pl.DeviceIdType.MESH,
            )
            r.start()
            l.start()
            if h == 0:
                compute_half(my, True)
                compute_half(my, False)
            else:
                compute_half((my - h) % N_DEV, True)
                compute_half((my + h) % N_DEV, False)
            r.wait()
            l.wait()

        compute_half((my + 1) % N_DEV, True)
        compute_half((my - 1) % N_DEV, False)
        for s in (0, 1):
            if pending[s] is not None:
                pending[s].wait()

    return pl.pallas_call(
        body,
        out_shape=jax.ShapeDtypeStruct((N_DEV * m_per, n_per), jnp.float32),
        in_specs=[
            pl.BlockSpec(memory_space=pltpu.VMEM),
            pl.BlockSpec(memory_space=pltpu.VMEM),
            pl.BlockSpec(memory_space=pltpu.SMEM),
            pl.BlockSpec(memory_space=pltpu.SMEM),
        ],
        out_specs=pl.BlockSpec(memory_space=pl.ANY),
        scratch_shapes=[
            pltpu.VMEM((N_DEV, m_per, k), jnp.int8),
            pltpu.VMEM((2, m_half, n_per), jnp.float32),
            pltpu.SemaphoreType.DMA((N_DEV - 1,)),
            pltpu.SemaphoreType.DMA((N_DEV - 1,)),
            pltpu.SemaphoreType.DMA((N_DEV - 1,)),
            pltpu.SemaphoreType.DMA((N_DEV - 1,)),
            pltpu.SemaphoreType.DMA((2,)),
        ],
        compiler_params=pltpu.CompilerParams(
            collective_id=0,
            vmem_limit_bytes=100 * 1024 * 1024,
        ),
    )(x, w_mat, scale_x, scale_w)
